# baseline (speedup 1.0000x reference)
import jax
import jax.numpy as jnp
from jax import lax
from jax.experimental import pallas as pl
from jax.experimental.pallas import tpu as pltpu
from jax.experimental.pallas import tpu_sc as plsc

N = 10000
E = 320000
D = 128
DH = D // 2
NP = 10240
RB = 2048
NC = 2
NS = 16
EPS = E // NS
K = 80
T = EPS // K
RPS = NP // NS


def _tc_transform_body(x_ref, w_ref, a_ref, h_ref, sa_ref):
    h = jnp.dot(x_ref[...], w_ref[...], preferred_element_type=jnp.float32)
    h_ref[0] = h[:, :DH]
    h_ref[1] = h[:, DH:]
    sa_ref[...] = jnp.dot(h, a_ref[...], preferred_element_type=jnp.float32)


def _tc_transform(x_p, w, a2):
    return pl.pallas_call(
        _tc_transform_body,
        grid=(NP // RB,),
        in_specs=[
            pl.BlockSpec((RB, D), lambda i: (i, 0)),
            pl.BlockSpec((D, D), lambda i: (0, 0)),
            pl.BlockSpec((D, 2), lambda i: (0, 0)),
        ],
        out_specs=[
            pl.BlockSpec((NC, RB, DH), lambda i: (0, i, 0)),
            pl.BlockSpec((RB, 2), lambda i: (i, 0)),
        ],
        out_shape=[
            jax.ShapeDtypeStruct((NC, NP, DH), jnp.float32),
            jax.ShapeDtypeStruct((NP, 2), jnp.float32),
        ],
    )(x_p, w, a2)


def _tc_mid_body(acc_ref, den_ref, b_ref, w_ref, a_ref, h_ref, sa_ref):
    num = jnp.concatenate([acc_ref[0], acc_ref[1]], axis=1)
    den = jnp.sum(den_ref[...], axis=0)
    out1 = num / (den + 1e-16)[:, None] + b_ref[...]
    hmid = jnp.maximum(out1, 0.0)
    h2 = jnp.dot(hmid, w_ref[...], preferred_element_type=jnp.float32)
    h_ref[0] = h2[:, :DH]
    h_ref[1] = h2[:, DH:]
    sa_ref[...] = jnp.dot(h2, a_ref[...], preferred_element_type=jnp.float32)


def _tc_mid(acc, den, b_2d, w, a2):
    return pl.pallas_call(
        _tc_mid_body,
        grid=(NP // RB,),
        in_specs=[
            pl.BlockSpec((NC, RB, DH), lambda i: (0, i, 0)),
            pl.BlockSpec((NC * NS, RB), lambda i: (0, i)),
            pl.BlockSpec((1, D), lambda i: (0, 0)),
            pl.BlockSpec((D, D), lambda i: (0, 0)),
            pl.BlockSpec((D, 2), lambda i: (0, 0)),
        ],
        out_specs=[
            pl.BlockSpec((NC, RB, DH), lambda i: (0, i, 0)),
            pl.BlockSpec((RB, 2), lambda i: (i, 0)),
        ],
        out_shape=[
            jax.ShapeDtypeStruct((NC, NP, DH), jnp.float32),
            jax.ShapeDtypeStruct((NP, 2), jnp.float32),
        ],
    )(acc, den, b_2d, w, a2)


def _tc_final_body(acc_ref, den_ref, b_ref, g_ref):
    i = pl.program_id(0)
    num = jnp.concatenate([acc_ref[0], acc_ref[1]], axis=1)
    den = jnp.sum(den_ref[...], axis=0)
    out2 = num / (den + 1e-16)[:, None] + b_ref[...]
    rows = i * RB + lax.broadcasted_iota(jnp.int32, (RB, 1), 0)
    out2 = jnp.where(rows < N, out2, 0.0)
    part = jnp.sum(out2, axis=0, keepdims=True)

    @pl.when(i == 0)
    def _():
        g_ref[...] = jnp.zeros_like(g_ref)

    g_ref[...] += part

    @pl.when(i == NP // RB - 1)
    def _():
        g_ref[...] = g_ref[...] * (1.0 / N)


def _tc_final(acc, den, b_2d):
    return pl.pallas_call(
        _tc_final_body,
        grid=(NP // RB,),
        in_specs=[
            pl.BlockSpec((NC, RB, DH), lambda i: (0, i, 0)),
            pl.BlockSpec((NC * NS, RB), lambda i: (0, i)),
            pl.BlockSpec((1, D), lambda i: (0, 0)),
        ],
        out_specs=pl.BlockSpec((1, D), lambda i: (0, 0)),
        out_shape=jax.ShapeDtypeStruct((1, D), jnp.float32),
    )(acc, den, b_2d)


def _sc_edge_body(h_hbm, as_hbm, ad_hbm, src_hbm, dst_hbm,
                  acc_out, den_out,
                  src_v, dst_v, asb, adb, eeb, idxb, denl, rows, zbuf, acc,
                  gsem, ssem):
    cid = lax.axis_index("c")
    sid = lax.axis_index("s")
    goff = cid * NP

    z16 = jnp.zeros((16,), jnp.float32)

    def _zrow(r, carry):
        for c in range(DH // 16):
            zbuf[r, pl.ds(c * 16, 16)] = z16
        return carry
    lax.fori_loop(0, 128, _zrow, None)

    def _zden(i, carry):
        denl[pl.ds(i * 16, 16)] = z16
        return carry
    lax.fori_loop(0, NP // 16, _zden, None)

    for r in range(RPS // 128):
        pltpu.sync_copy(zbuf, acc.at[pl.ds(sid * RPS + r * 128, 128)])
    plsc.subcore_barrier()

    pltpu.sync_copy(src_hbm.at[sid], src_v)
    pltpu.sync_copy(dst_hbm.at[sid], dst_v)
    pltpu.sync_copy(as_hbm, asb)
    pltpu.sync_copy(ad_hbm, adb)

    def _ee_tile(t, b):
        for c in range(K // 16):
            sv = src_v[t, pl.ds(c * 16, 16)]
            dv = dst_v[t, pl.ds(c * 16, 16)]
            e = plsc.load_gather(asb, [sv]) + plsc.load_gather(adb, [dv])
            e = jnp.where(e >= 0.0, e, 0.2 * e)
            ee = jnp.exp(e)
            eeb[b, pl.ds(c * 16, 16)] = ee
            idxb[b, pl.ds(c * 16, 16)] = sv + goff
            m = jnp.full((16,), cid == (c % 2), jnp.bool_)
            plsc.addupdate_scatter(denl, [dv], ee, mask=m)

    _ee_tile(0, 0)
    pltpu.async_copy(h_hbm.at[idxb.at[0]], rows.at[0], gsem)

    def _outer(g, carry):
        for b in range(2):
            t = 2 * g + b
            nb = 1 - b

            @pl.when(t >= 1)
            def _():
                pltpu.make_async_copy(rows.at[nb], acc.at[dst_v.at[t]],
                                      ssem).wait()

            @pl.when(t + 1 < T)
            def _():
                _ee_tile(t + 1, nb)
                pltpu.async_copy(h_hbm.at[idxb.at[nb]], rows.at[nb], gsem)

            pltpu.make_async_copy(h_hbm.at[idxb.at[b]], rows.at[b],
                                  gsem).wait()

            @plsc.parallel_loop(0, K, unroll=16)
            def _scale(j):
                s = plsc.load_gather(eeb.at[b], [jnp.full((16,), j, jnp.int32)])
                for c in range(DH // 16):
                    rows[b, j, pl.ds(c * 16, 16)] = (
                        rows[b, j, pl.ds(c * 16, 16)] * s)
            pltpu.async_copy(rows.at[b], acc.at[dst_v.at[t]], ssem, add=True)
        return carry
    lax.fori_loop(0, T // 2, _outer, None)
    pltpu.make_async_copy(rows.at[1], acc.at[dst_v.at[T - 1]], ssem).wait()
    plsc.subcore_barrier()

    pltpu.sync_copy(acc.at[pl.ds(sid * RPS, RPS)],
                    acc_out.at[cid, pl.ds(sid * RPS, RPS)])

    pltpu.sync_copy(denl, den_out.at[cid * NS + sid])


_sc_edge = pl.kernel(
    _sc_edge_body,
    out_type=[
        jax.ShapeDtypeStruct((NC, NP, DH), jnp.float32),
        jax.ShapeDtypeStruct((NC * NS, NP), jnp.float32),
    ],
    mesh=plsc.VectorSubcoreMesh(core_axis_name="c", subcore_axis_name="s"),
    compiler_params=pltpu.CompilerParams(needs_layout_passes=False,
                                         use_tc_tiling_on_sc=False),
    scratch_types=[
        pltpu.VMEM((T, K), jnp.int32),
        pltpu.VMEM((T, K), jnp.int32),
        pltpu.VMEM((NP,), jnp.float32),
        pltpu.VMEM((NP,), jnp.float32),
        pltpu.VMEM((2, K), jnp.float32),
        pltpu.VMEM((2, K), jnp.int32),
        pltpu.VMEM((NP,), jnp.float32),
        pltpu.VMEM((2, K, DH), jnp.float32),
        pltpu.VMEM((128, DH), jnp.float32),
        pltpu.VMEM_SHARED((NP, DH), jnp.float32),
        pltpu.SemaphoreType.DMA,
        pltpu.SemaphoreType.DMA,
    ],
)


def kernel(x, edge_index, W1, a_src1, a_dst1, b1, W2, a_src2, a_dst2, b2):
    x_p = jnp.zeros((NP, D), jnp.float32).at[:N].set(x)
    a1 = jnp.stack([a_src1, a_dst1], axis=1)
    a2 = jnp.stack([a_src2, a_dst2], axis=1)
    src3d = edge_index[0].reshape(NS, T, K)
    dst3d = edge_index[1].reshape(NS, T, K)

    h1, sa1 = _tc_transform(x_p, W1, a1)
    acc1, den1 = _sc_edge(h1.reshape(NC * NP, DH), sa1[:, 0], sa1[:, 1],
                          src3d, dst3d)
    h2, sa2 = _tc_mid(acc1, den1, b1.reshape(1, D), W2, a2)
    acc2, den2 = _sc_edge(h2.reshape(NC * NP, DH), sa2[:, 0], sa2[:, 1],
                          src3d, dst3d)
    return _tc_final(acc2, den2, b2.reshape(1, D))


# speedup vs baseline: 1.0609x; 1.0609x over previous
import jax
import jax.numpy as jnp
import numpy as np
from jax import lax
from jax.experimental import pallas as pl
from jax.experimental.pallas import tpu as pltpu
from jax.experimental.pallas import tpu_sc as plsc

N = 10000
E = 320000
D = 128
DH = D // 2
NP = 10240
RB = 2048
NC = 2
NS = 16
EPS = E // NS
K = 80
T = EPS // K
RPS = NP // NS

_PERM = np.zeros(D, np.int32)
for _h in range(2):
    for _c in range(2):
        for _q in range(32):
            _PERM[_h * 64 + _c * 32 + _q] = (
                _h * 64 + _c * 32 + 16 * (_q % 2) + _q // 2)


def _tc_transform_body(x_ref, w_ref, a_ref, h_ref, sa_ref):
    h = jnp.dot(x_ref[...], w_ref[...], preferred_element_type=jnp.float32)
    hb = h.astype(jnp.bfloat16)
    h_ref[0] = hb[:, :DH]
    h_ref[1] = hb[:, DH:]
    sa_ref[...] = jnp.dot(h, a_ref[...], preferred_element_type=jnp.float32)


def _tc_transform(x_p, w, a2):
    return pl.pallas_call(
        _tc_transform_body,
        grid=(NP // RB,),
        in_specs=[
            pl.BlockSpec((RB, D), lambda i: (i, 0)),
            pl.BlockSpec((D, D), lambda i: (0, 0)),
            pl.BlockSpec((D, 2), lambda i: (0, 0)),
        ],
        out_specs=[
            pl.BlockSpec((NC, RB, DH), lambda i: (0, i, 0)),
            pl.BlockSpec((RB, 2), lambda i: (i, 0)),
        ],
        out_shape=[
            jax.ShapeDtypeStruct((NC, NP, DH), jnp.bfloat16),
            jax.ShapeDtypeStruct((NP, 2), jnp.float32),
        ],
    )(x_p, w, a2)


def _tc_mid_body(acc_ref, den_ref, b_ref, w_ref, a_ref, h_ref, sa_ref):
    num = jnp.concatenate([acc_ref[0], acc_ref[1]], axis=1)
    den = jnp.sum(den_ref[...], axis=0)
    out1 = num / (den + 1e-16)[:, None] + b_ref[...]
    hmid = jnp.maximum(out1, 0.0)
    h2 = jnp.dot(hmid, w_ref[...], preferred_element_type=jnp.float32)
    hb = h2.astype(jnp.bfloat16)
    h_ref[0] = hb[:, :DH]
    h_ref[1] = hb[:, DH:]
    sa_ref[...] = jnp.dot(h2, a_ref[...], preferred_element_type=jnp.float32)


def _tc_mid(acc, den, b_2d, w, a2):
    return pl.pallas_call(
        _tc_mid_body,
        grid=(NP // RB,),
        in_specs=[
            pl.BlockSpec((NC, RB, DH), lambda i: (0, i, 0)),
            pl.BlockSpec((NC * NS, RB), lambda i: (0, i)),
            pl.BlockSpec((1, D), lambda i: (0, 0)),
            pl.BlockSpec((D, D), lambda i: (0, 0)),
            pl.BlockSpec((D, 2), lambda i: (0, 0)),
        ],
        out_specs=[
            pl.BlockSpec((NC, RB, DH), lambda i: (0, i, 0)),
            pl.BlockSpec((RB, 2), lambda i: (i, 0)),
        ],
        out_shape=[
            jax.ShapeDtypeStruct((NC, NP, DH), jnp.bfloat16),
            jax.ShapeDtypeStruct((NP, 2), jnp.float32),
        ],
    )(acc, den, b_2d, w, a2)


def _tc_final_body(acc_ref, den_ref, b_ref, g_ref):
    i = pl.program_id(0)
    num = jnp.concatenate([acc_ref[0], acc_ref[1]], axis=1)
    den = jnp.sum(den_ref[...], axis=0)
    out2 = num / (den + 1e-16)[:, None] + b_ref[...]
    rows = i * RB + lax.broadcasted_iota(jnp.int32, (RB, 1), 0)
    out2 = jnp.where(rows < N, out2, 0.0)
    part = jnp.sum(out2, axis=0, keepdims=True)

    @pl.when(i == 0)
    def _():
        g_ref[...] = jnp.zeros_like(g_ref)

    g_ref[...] += part

    @pl.when(i == NP // RB - 1)
    def _():
        g_ref[...] = g_ref[...] * (1.0 / N)


def _tc_final(acc, den, b_2d):
    return pl.pallas_call(
        _tc_final_body,
        grid=(NP // RB,),
        in_specs=[
            pl.BlockSpec((NC, RB, DH), lambda i: (0, i, 0)),
            pl.BlockSpec((NC * NS, RB), lambda i: (0, i)),
            pl.BlockSpec((1, D), lambda i: (0, 0)),
        ],
        out_specs=pl.BlockSpec((1, D), lambda i: (0, 0)),
        out_shape=jax.ShapeDtypeStruct((1, D), jnp.float32),
    )(acc, den, b_2d)


def _sc_edge_body(h_hbm, as_hbm, ad_hbm, src_hbm, dst_hbm,
                  acc_out, den_out,
                  src_v, dst_v, asb, adb, eeb, idxb, denl, rbf, rows,
                  acc, gsem, ssem):
    cid = lax.axis_index("c")
    sid = lax.axis_index("s")
    goff = cid * NP

    z16 = jnp.zeros((16,), jnp.float32)

    def _zrow(r, carry):
        for c in range(DH // 16):
            rows[0, r, pl.ds(c * 16, 16)] = z16
        return carry
    lax.fori_loop(0, K, _zrow, None)

    def _zden(i, carry):
        denl[pl.ds(i * 16, 16)] = z16
        return carry
    lax.fori_loop(0, NP // 16, _zden, None)

    for r in range(RPS // K):
        pltpu.sync_copy(rows.at[0], acc.at[pl.ds(sid * RPS + r * K, K)])
    plsc.subcore_barrier()

    pltpu.sync_copy(src_hbm.at[sid], src_v)
    pltpu.sync_copy(dst_hbm.at[sid], dst_v)
    pltpu.sync_copy(as_hbm, asb)
    pltpu.sync_copy(ad_hbm, adb)

    def _ee_tile(t, b):
        for c in range(K // 16):
            sv = src_v[t, pl.ds(c * 16, 16)]
            dv = dst_v[t, pl.ds(c * 16, 16)]
            e = plsc.load_gather(asb, [sv]) + plsc.load_gather(adb, [dv])
            e = jnp.where(e >= 0.0, e, 0.2 * e)
            ee = jnp.exp(e)
            eeb[b, pl.ds(c * 16, 16)] = ee
            idxb[b, pl.ds(c * 16, 16)] = sv + goff
            m = jnp.full((16,), cid == (c % 2), jnp.bool_)
            plsc.addupdate_scatter(denl, [dv], ee, mask=m)

    _ee_tile(0, 0)
    pltpu.async_copy(h_hbm.at[idxb.at[0]], rbf.at[0], gsem)

    def _outer(g, carry):
        for b in range(2):
            t = 2 * g + b
            nb = 1 - b

            @pl.when(t >= 1)
            def _():
                pltpu.make_async_copy(rows.at[nb], acc.at[dst_v.at[t]],
                                      ssem).wait()

            @pl.when(t + 1 < T)
            def _():
                _ee_tile(t + 1, nb)
                pltpu.async_copy(h_hbm.at[idxb.at[nb]], rbf.at[nb], gsem)

            pltpu.make_async_copy(h_hbm.at[idxb.at[b]], rbf.at[b],
                                  gsem).wait()

            @plsc.parallel_loop(0, K, unroll=8)
            def _scale(j):
                s = plsc.load_gather(eeb.at[b], [jnp.full((16,), j, jnp.int32)])
                for c in range(DH // 32):
                    v = rbf[b, j, pl.ds(c * 32, 32)]
                    pa, pb = plsc.unpack(v, format=plsc.PackFormat.INTERLEAVED)
                    rows[b, j, pl.ds(c * 32, 16)] = pa * s
                    rows[b, j, pl.ds(c * 32 + 16, 16)] = pb * s
            pltpu.async_copy(rows.at[b], acc.at[dst_v.at[t]], ssem, add=True)
        return carry
    lax.fori_loop(0, T // 2, _outer, None)
    pltpu.make_async_copy(rows.at[1], acc.at[dst_v.at[T - 1]], ssem).wait()
    plsc.subcore_barrier()

    pltpu.sync_copy(acc.at[pl.ds(sid * RPS, RPS)],
                    acc_out.at[cid, pl.ds(sid * RPS, RPS)])

    pltpu.sync_copy(denl, den_out.at[cid * NS + sid])


_sc_edge = pl.kernel(
    _sc_edge_body,
    out_type=[
        jax.ShapeDtypeStruct((NC, NP, DH), jnp.float32),
        jax.ShapeDtypeStruct((NC * NS, NP), jnp.float32),
    ],
    mesh=plsc.VectorSubcoreMesh(core_axis_name="c", subcore_axis_name="s"),
    compiler_params=pltpu.CompilerParams(needs_layout_passes=False,
                                         use_tc_tiling_on_sc=False),
    scratch_types=[
        pltpu.VMEM((T, K), jnp.int32),
        pltpu.VMEM((T, K), jnp.int32),
        pltpu.VMEM((NP,), jnp.float32),
        pltpu.VMEM((NP,), jnp.float32),
        pltpu.VMEM((2, K), jnp.float32),
        pltpu.VMEM((2, K), jnp.int32),
        pltpu.VMEM((NP,), jnp.float32),
        pltpu.VMEM((2, K, DH), jnp.bfloat16),
        pltpu.VMEM((2, K, DH), jnp.float32),
        pltpu.VMEM_SHARED((NP, DH), jnp.float32),
        pltpu.SemaphoreType.DMA,
        pltpu.SemaphoreType.DMA,
    ],
)


def kernel(x, edge_index, W1, a_src1, a_dst1, b1, W2, a_src2, a_dst2, b2):
    x_p = jnp.zeros((NP, D), jnp.float32).at[:N].set(x)
    prm = jnp.asarray(_PERM)
    W1p = W1[:, prm]
    W2p = W2[:, prm]
    a1 = jnp.stack([a_src1[prm], a_dst1[prm]], axis=1)
    a2 = jnp.stack([a_src2[prm], a_dst2[prm]], axis=1)
    src3d = edge_index[0].reshape(NS, T, K)
    dst3d = edge_index[1].reshape(NS, T, K)

    h1, sa1 = _tc_transform(x_p, W1p, a1)
    acc1, den1 = _sc_edge(h1.reshape(NC * NP, DH), sa1[:, 0], sa1[:, 1],
                          src3d, dst3d)
    h2, sa2 = _tc_mid(acc1, den1, b1.reshape(1, D), W2p, a2)
    acc2, den2 = _sc_edge(h2.reshape(NC * NP, DH), sa2[:, 0], sa2[:, 1],
                          src3d, dst3d)
    return _tc_final(acc2, den2, b2.reshape(1, D))
